# Initial kernel scaffold; baseline (speedup 1.0000x reference)
import jax
import jax.numpy as jnp
from jax.experimental import pallas as pl


def kernel(x, grid_value_param, grid_feature_param):
    raise NotImplementedError("write your pallas kernel here")


# speedup vs baseline: 1.6079x; 1.6079x over previous
import functools

import jax
import jax.numpy as jnp
from jax import lax
from jax.experimental import pallas as pl
from jax.experimental.pallas import tpu as pltpu
from jax.experimental.pallas import tpu_sc as plsc

N_GRID = 64
N1 = N_GRID + 1
V = N1 * N1 * N1
D = 128
B = 262144
L = 16

NC = 2
NS = 16
NW = NC * NS
PT = B // NW
C = 64
NCHUNK = PT // C

_OFFS = tuple(ox * (N1 * N1) + oy * N1 + oz
              for ox in (0, 1) for oy in (0, 1) for oz in (0, 1))


def _body(xT, valt, feat, outv_hbm, outf_hbm,
          xv, idxb, wb, rows, vrows, outf, outv, sem):
    wid = lax.axis_index("s") * NC + lax.axis_index("c")
    base = wid * PT

    def chunk(i, carry):
        t = base + i * C
        for d in range(3):
            pltpu.sync_copy(xT.at[pl.ds(d * B + t, C)], xv.at[pl.ds(d * C, C)])

        for g in range(C // L):
            s = g * L
            px = (xv[pl.ds(s, L)] + 1.0) * 32.0
            py = (xv[pl.ds(C + s, L)] + 1.0) * 32.0
            pz = (xv[pl.ds(2 * C + s, L)] + 1.0) * 32.0
            ix = px.astype(jnp.int32)
            iy = py.astype(jnp.int32)
            iz = pz.astype(jnp.int32)
            fx = px - ix.astype(jnp.float32)
            fy = py - iy.astype(jnp.float32)
            fz = pz - iz.astype(jnp.float32)
            b0 = ix * (N1 * N1) + iy * N1 + iz
            cidx = 0
            for ox in (0, 1):
                wx = fx if ox else 1.0 - fx
                for oy in (0, 1):
                    wxy = wx * (fy if oy else 1.0 - fy)
                    for oz in (0, 1):
                        w = wxy * (fz if oz else 1.0 - fz)
                        idxb[cidx, pl.ds(s, L)] = b0 + _OFFS[cidx]
                        wb[pl.ds(cidx * C + s, L)] = w
                        cidx += 1

        cps = []
        for c in range(8):
            cps.append(pltpu.async_copy(
                feat.at[idxb.at[c]], rows.at[pl.ds(c * C, C)], sem))
        for c in range(8):
            cps.append(pltpu.async_copy(
                valt.at[idxb.at[c]], vrows.at[c], sem))
        for cp in cps:
            cp.wait()

        for g in range(C // L):
            s = g * L
            acc = wb[pl.ds(s, L)] * vrows[0, pl.ds(s, L)]
            for c in range(1, 8):
                acc = acc + wb[pl.ds(c * C + s, L)] * vrows[c, pl.ds(s, L)]
            outv[pl.ds(s, L)] = acc

        def pt(j, carry2):
            jv = jnp.full((L,), j, dtype=jnp.int32)
            acc = [None] * (D // L)
            for c in range(8):
                ws = plsc.load_gather(wb, [jv + (c * C)])
                r = c * C + j
                for k in range(D // L):
                    rk = rows[r, pl.ds(k * L, L)]
                    if c == 0:
                        acc[k] = ws * rk
                    else:
                        acc[k] = acc[k] + ws * rk
            for k in range(D // L):
                outf[j, pl.ds(k * L, L)] = acc[k]
            return carry2

        lax.fori_loop(0, C, pt, 0, unroll=2)

        pltpu.sync_copy(outf, outf_hbm.at[pl.ds(t, C)])
        pltpu.sync_copy(outv, outv_hbm.at[pl.ds(t, C)])
        return carry

    lax.fori_loop(0, NCHUNK, chunk, 0)


_sc_call = pl.kernel(
    _body,
    out_type=(
        jax.ShapeDtypeStruct((B,), jnp.float32),
        jax.ShapeDtypeStruct((B, D), jnp.float32),
    ),
    mesh=plsc.VectorSubcoreMesh(core_axis_name="c", subcore_axis_name="s"),
    compiler_params=pltpu.CompilerParams(needs_layout_passes=False),
    scratch_types=(
        pltpu.VMEM((3 * C,), jnp.float32),
        pltpu.VMEM((8, C), jnp.int32),
        pltpu.VMEM((8 * C,), jnp.float32),
        pltpu.VMEM((8 * C, D), jnp.float32),
        pltpu.VMEM((8, C), jnp.float32),
        pltpu.VMEM((C, D), jnp.float32),
        pltpu.VMEM((C,), jnp.float32),
        pltpu.SemaphoreType.DMA,
    ),
)


@jax.jit
def kernel(x, grid_value_param, grid_feature_param):
    xT = x.T.reshape(-1)
    valt = grid_value_param.reshape(-1)
    outv, outf = _sc_call(xT, valt, grid_feature_param)
    return outv.reshape(B, 1), outf


# speedup vs baseline: 3.2232x; 2.0046x over previous
import jax
import jax.numpy as jnp
from jax import lax
from jax.experimental import pallas as pl
from jax.experimental.pallas import tpu as pltpu
from jax.experimental.pallas import tpu_sc as plsc

N_GRID = 64
N1 = N_GRID + 1
V = N1 * N1 * N1
D = 128
B = 262144
L = 16

NC = 2
NS = 16
NW = NC * NS
PT = B // NW
C = 32
NCHUNK = PT // C

_OFFS = tuple(ox * (N1 * N1) + oy * N1 + oz
              for ox in (0, 1) for oy in (0, 1) for oz in (0, 1))


def _body(xT, valt, feat, outv_hbm, outf_hbm, xv, *bufs_flat):
    semg = bufs_flat[-4:-2]
    semo = bufs_flat[-2:]
    bufs = (bufs_flat[0:6], bufs_flat[6:12])

    wid = lax.axis_index("s") * NC + lax.axis_index("c")
    base = wid * PT

    for d in range(3):
        pltpu.sync_copy(xT.at[pl.ds(d * B + base, PT)],
                        xv.at[pl.ds(d * PT, PT)])

    def compute_idx(i, idxb, wb):
        off = i * C
        for g in range(C // L):
            s = off + g * L
            px = (xv[pl.ds(s, L)] + 1.0) * 32.0
            py = (xv[pl.ds(PT + s, L)] + 1.0) * 32.0
            pz = (xv[pl.ds(2 * PT + s, L)] + 1.0) * 32.0
            ix = px.astype(jnp.int32)
            iy = py.astype(jnp.int32)
            iz = pz.astype(jnp.int32)
            fx = px - ix.astype(jnp.float32)
            fy = py - iy.astype(jnp.float32)
            fz = pz - iz.astype(jnp.float32)
            b0 = ix * (N1 * N1) + iy * N1 + iz
            cidx = 0
            for ox in (0, 1):
                wx = fx if ox else 1.0 - fx
                for oy in (0, 1):
                    wxy = wx * (fy if oy else 1.0 - fy)
                    for oz in (0, 1):
                        w = wxy * (fz if oz else 1.0 - fz)
                        idxb[cidx, pl.ds(g * L, L)] = b0 + _OFFS[cidx]
                        wb[pl.ds(cidx * C + g * L, L)] = w
                        cidx += 1

    def fire_gathers(idxb, rows, vrows, sem):
        for c in range(8):
            pltpu.async_copy(feat.at[idxb.at[c]],
                             rows.at[pl.ds(c * C, C)], sem)
            pltpu.async_copy(valt.at[idxb.at[c]], vrows.at[c], sem)

    def wait_gathers(idxb, rows, vrows, sem):
        for c in range(8):
            pltpu.make_async_copy(feat.at[idxb.at[c]],
                                  rows.at[pl.ds(c * C, C)], sem).wait()
            pltpu.make_async_copy(valt.at[idxb.at[c]], vrows.at[c],
                                  sem).wait()

    def accumulate(wb, rows, vrows, outf, outv):
        for g in range(C // L):
            s = g * L
            acc = wb[pl.ds(s, L)] * vrows[0, pl.ds(s, L)]
            for c in range(1, 8):
                acc = acc + wb[pl.ds(c * C + s, L)] * vrows[c, pl.ds(s, L)]
            outv[pl.ds(s, L)] = acc

        def pt(j, carry2):
            jv = jnp.full((L,), j, dtype=jnp.int32)
            acc = [None] * (D // L)
            for c in range(8):
                ws = plsc.load_gather(wb, [jv + (c * C)])
                r = c * C + j
                for k in range(D // L):
                    rk = rows[r, pl.ds(k * L, L)]
                    if c == 0:
                        acc[k] = ws * rk
                    else:
                        acc[k] = acc[k] + ws * rk
            for k in range(D // L):
                outf[j, pl.ds(k * L, L)] = acc[k]
            return carry2

        lax.fori_loop(0, C, pt, 0, unroll=2)

    def fire_out(i, outf, outv, sem):
        t = base + i * C
        pltpu.async_copy(outf, outf_hbm.at[pl.ds(t, C)], sem)
        pltpu.async_copy(outv, outv_hbm.at[pl.ds(t, C)], sem)

    def wait_out(outf, outv, sem):
        pltpu.make_async_copy(outf, outf_hbm.at[pl.ds(base, C)], sem).wait()
        pltpu.make_async_copy(outv, outv_hbm.at[pl.ds(base, C)], sem).wait()

    idxb0, wb0, rows0, vrows0, _, _ = bufs[0]
    compute_idx(0, idxb0, wb0)
    fire_gathers(idxb0, rows0, vrows0, semg[0])

    def body2(k, carry):
        for p in (0, 1):
            i = 2 * k + p
            q = 1 - p
            idxb, wb, rows, vrows, outf, outv = bufs[p]
            idxbq, wbq, rowsq, vrowsq, _, _ = bufs[q]

            @pl.when(i + 1 < NCHUNK)
            def _prefetch():
                compute_idx(i + 1, idxbq, wbq)
                fire_gathers(idxbq, rowsq, vrowsq, semg[q])

            wait_gathers(idxb, rows, vrows, semg[p])

            @pl.when(i >= 2)
            def _drain_out():
                wait_out(outf, outv, semo[p])

            accumulate(wb, rows, vrows, outf, outv)
            fire_out(i, outf, outv, semo[p])
        return carry

    lax.fori_loop(0, NCHUNK // 2, body2, 0)

    for p in (0, 1):
        _, _, _, _, outf, outv = bufs[p]
        wait_out(outf, outv, semo[p])


def _parity_bufs():
    return (
        pltpu.VMEM((8, C), jnp.int32),
        pltpu.VMEM((8 * C,), jnp.float32),
        pltpu.VMEM((8 * C, D), jnp.float32),
        pltpu.VMEM((8, C), jnp.float32),
        pltpu.VMEM((C, D), jnp.float32),
        pltpu.VMEM((C,), jnp.float32),
    )


_sc_call = pl.kernel(
    _body,
    out_type=(
        jax.ShapeDtypeStruct((B,), jnp.float32),
        jax.ShapeDtypeStruct((B, D), jnp.float32),
    ),
    mesh=plsc.VectorSubcoreMesh(core_axis_name="c", subcore_axis_name="s"),
    compiler_params=pltpu.CompilerParams(needs_layout_passes=False),
    scratch_types=(
        pltpu.VMEM((3 * PT,), jnp.float32),
        *_parity_bufs(),
        *_parity_bufs(),
        pltpu.SemaphoreType.DMA,
        pltpu.SemaphoreType.DMA,
        pltpu.SemaphoreType.DMA,
        pltpu.SemaphoreType.DMA,
    ),
)


@jax.jit
def kernel(x, grid_value_param, grid_feature_param):
    xT = x.T.reshape(-1)
    valt = grid_value_param.reshape(-1)
    outv, outf = _sc_call(xT, valt, grid_feature_param)
    return outv.reshape(B, 1), outf
